# Initial kernel scaffold; baseline (speedup 1.0000x reference)
import jax
import jax.numpy as jnp
from jax.experimental import pallas as pl


def kernel(x, pro_emb, params, pro_emb_batch):
    raise NotImplementedError("write your pallas kernel here")


# speedup vs baseline: 18.6370x; 18.6370x over previous
import functools
import math

import jax
import jax.numpy as jnp
from jax.experimental import pallas as pl
from jax.experimental.pallas import tpu as pltpu

_NEG = -1e30


def _b16(x):
    return x.astype(jnp.bfloat16).astype(jnp.float32)


def _bdot(a, b):
    return jnp.dot(a.astype(jnp.bfloat16), b.astype(jnp.bfloat16),
                   preferred_element_type=jnp.float32)


def _pick_rows(nres: int) -> int:
    best = 0
    for r in range(1, min(nres, 1024) + 1):
        if nres % r == 0 and (r % 8 == 0 or best == 0):
            best = r
    return best if best else nres


def _att_kernel(nblk, layer, pe_ref, seg_ref, w2t_ref, wkt_ref, bk_ref,
                w1_ref, out_ref, m_ref, s_ref, o_ref):
    i = pl.program_id(0)
    bsz = o_ref.shape[1]

    @pl.when(i == 0)
    def _init():
        m_ref[...] = jnp.full(m_ref.shape, _NEG, jnp.float32)
        s_ref[...] = jnp.zeros(s_ref.shape, jnp.float32)
        o_ref[...] = jnp.zeros(o_ref.shape, jnp.float32)

    pe = pe_ref[...]
    prot = jnp.maximum(_bdot(pe, w2t_ref[...]), 0.0)
    rows = prot.shape[0]
    seg_row = seg_ref[0]
    onehot = (seg_row == jax.lax.broadcasted_iota(
        jnp.int32, (bsz, rows), 0)).astype(jnp.float32)

    es, alphas = [], []
    for l in range(layer):
        k = jnp.maximum(_bdot(prot, wkt_ref[l]) + bk_ref[l], 0.0)
        t = jnp.sum(_b16(k) * w1_ref[l], axis=1, keepdims=True)
        m_old = m_ref[l]
        m_new = jnp.maximum(m_old, jnp.max(t))
        alphas.append(jnp.exp(m_old - m_new))
        e = jnp.exp(t - m_new)
        es.append(e)
        contrib = jax.lax.dot_general(
            onehot, k * e, (((1,), (0,)), ((), ())),
            preferred_element_type=jnp.float32)
        o_ref[l] = alphas[l] * o_ref[l] + contrib
        m_ref[l] = m_new
    s_all = jax.lax.dot_general(
        onehot, jnp.concatenate(es, axis=1), (((1,), (0,)), ((), ())),
        preferred_element_type=jnp.float32)
    s_ref[...] = jnp.concatenate(alphas, axis=1) * s_ref[...] + s_all

    @pl.when(i == nblk - 1)
    def _fin():
        for l in range(layer):
            out_ref[l] = o_ref[l] / (s_ref[:, l:l + 1] + 1e-16)


def _ln(x, g, b, eps=1e-5):
    m = jnp.mean(x, axis=-1, keepdims=True)
    xc = x - m
    v = jnp.mean(xc * xc, axis=-1, keepdims=True)
    return xc * jax.lax.rsqrt(v + eps) * g + b


def _gelu(x):
    return 0.5 * x * (1.0 + jax.lax.erf(x / math.sqrt(2.0)))


def _softmax_pool(q, keys, vals, hid):
    scale = 1.0 / math.sqrt(hid)
    q16 = _b16(q)
    att = [jnp.sum(q16 * _b16(kk), axis=1, keepdims=True) * scale
           for kk in keys]
    mx = att[0]
    for a in att[1:]:
        mx = jnp.maximum(mx, a)
    es = [jnp.exp(a - mx) for a in att]
    den = es[0]
    for ee in es[1:]:
        den = den + ee
    ws = [ee / den for ee in es]
    acc = _b16(ws[0]) * _b16(vals[0])
    for ww, vv in zip(ws[1:], vals[1:]):
        acc = acc + _b16(ww) * _b16(vv)
    return acc


def _head_kernel(npatch, layer, hid,
                 x_ref, ap_ref, fpg_ref, fpb_ref, fpw_ref, fpbias_ref,
                 gdg_ref, gdb_ref, gd1w_ref, gd1b_ref, gd2w_ref, gd2b_ref,
                 gpw_ref, gpb_ref, qw_ref, qb_ref, lndg_ref, lndb_ref,
                 kdw_ref, kdb_ref, lnpg_ref, lnpb_ref, kpw_ref, kpb_ref,
                 l1w_ref, l1b_ref, l2w_ref, l2b_ref, l3w_ref, l3b_ref,
                 pr1_ref, pr2_ref, out_ref):
    x = x_ref[...]
    dtok = _bdot(_ln(x, fpg_ref[...], fpb_ref[...]),
                 fpw_ref[...]) + fpbias_ref[...]
    d_toks, kd = [], []
    for n in range(npatch):
        dn = _ln(dtok[:, n * hid:(n + 1) * hid], lndg_ref[...], lndb_ref[...])
        d_toks.append(dn)
        kd.append(_bdot(dn, kdw_ref[...]) + kdb_ref[...])
    g = _bdot(_ln(x, gdg_ref[...], gdb_ref[...]), gd1w_ref[...]) + gd1b_ref[...]
    g_d = _bdot(_gelu(g), gd2w_ref[...]) + gd2b_ref[...]
    ap_mean = ap_ref[0]
    for l in range(1, layer):
        ap_mean = ap_mean + ap_ref[l]
    ap_mean = ap_mean * (1.0 / layer)
    g_p = _gelu(_bdot(ap_mean, gpw_ref[...]) + gpb_ref[...])
    q_all = (_bdot(g_d, qw_ref[:hid]) + _bdot(g_p, qw_ref[hid:])
             + qb_ref[...])
    pts, kp = [], []
    for l in range(layer):
        pt = _ln(ap_ref[l], lnpg_ref[...], lnpb_ref[...])
        pts.append(pt)
        kp.append(_bdot(pt, kpw_ref[...]) + kpb_ref[...])
    vp = _softmax_pool(q_all, kp, pts, hid)
    vd = _softmax_pool(q_all, kd, d_toks, hid)
    z = _bdot(vd, l1w_ref[:hid]) + _bdot(vp, l1w_ref[hid:]) + l1b_ref[...]
    z = jnp.where(z >= 0, z, pr1_ref[...] * z)
    z = _bdot(z, l2w_ref[...]) + l2b_ref[...]
    z = jnp.where(z >= 0, z, pr2_ref[...] * z)
    out_ref[...] = jnp.sum(_b16(z) * _b16(l3w_ref[...]), axis=1,
                           keepdims=True) + l3b_ref[...]


def kernel(x, pro_emb, params, pro_emb_batch):
    p = params
    bsz, _ = x.shape
    nres, pdim = pro_emb.shape
    hid = p['prej2.W'].shape[0]
    layer = len([k for k in p if k.startswith('att') and k.endswith('.q.W')])
    npatch = p['ccfm.fp_proj.W'].shape[0] // hid
    rows = _pick_rows(nres)
    nblk = nres // rows

    w2t = p['prej2.W'].T.astype(jnp.bfloat16)
    wkt = jnp.stack([p['att%d.k.W' % l].T for l in range(layer)]
                    ).astype(jnp.bfloat16)
    bk = jnp.stack([p['att%d.k.b' % l][None, :] for l in range(layer)])
    w1 = jnp.stack([p['att%d.merge.W' % l][:, :hid] for l in range(layer)]
                   ).astype(jnp.bfloat16).astype(jnp.float32)
    seg3 = pro_emb_batch.reshape(nblk, 1, rows)

    att = pl.pallas_call(
        functools.partial(_att_kernel, nblk, layer),
        grid=(nblk,),
        in_specs=[
            pl.BlockSpec((rows, pdim), lambda i: (i, 0)),
            pl.BlockSpec((1, 1, rows), lambda i: (i, 0, 0)),
            pl.BlockSpec((pdim, hid), lambda i: (0, 0)),
            pl.BlockSpec((layer, hid, hid), lambda i: (0, 0, 0)),
            pl.BlockSpec((layer, 1, hid), lambda i: (0, 0, 0)),
            pl.BlockSpec((layer, 1, hid), lambda i: (0, 0, 0)),
        ],
        out_specs=pl.BlockSpec((layer, bsz, hid), lambda i: (0, 0, 0)),
        out_shape=jax.ShapeDtypeStruct((layer, bsz, hid), jnp.float32),
        scratch_shapes=[
            pltpu.VMEM((layer, 1, 1), jnp.float32),
            pltpu.VMEM((bsz, layer), jnp.float32),
            pltpu.VMEM((layer, bsz, hid), jnp.float32),
        ],
        compiler_params=pltpu.CompilerParams(
            dimension_semantics=("arbitrary",)),
    )(pro_emb, seg3, w2t, wkt, bk, w1)

    out = pl.pallas_call(
        functools.partial(_head_kernel, npatch, layer, hid),
        out_shape=jax.ShapeDtypeStruct((bsz, 1), jnp.float32),
    )(
        x, att,
        p['ccfm.fp_ln.g'][None, :], p['ccfm.fp_ln.b'][None, :],
        p['ccfm.fp_proj.W'].T, p['ccfm.fp_proj.b'][None, :],
        p['ccfm.gd_ln.g'][None, :], p['ccfm.gd_ln.b'][None, :],
        p['ccfm.gd1.W'].T, p['ccfm.gd1.b'][None, :],
        p['ccfm.gd2.W'].T, p['ccfm.gd2.b'][None, :],
        p['ccfm.gp.W'].T, p['ccfm.gp.b'][None, :],
        p['ccfm.q.W'].T, p['ccfm.q.b'][None, :],
        p['ccfm.ln_d.g'][None, :], p['ccfm.ln_d.b'][None, :],
        p['ccfm.k_d.W'].T, p['ccfm.k_d.b'][None, :],
        p['ccfm.ln_p.g'][None, :], p['ccfm.ln_p.b'][None, :],
        p['ccfm.k_p.W'].T, p['ccfm.k_p.b'][None, :],
        p['out.l1.W'].T, p['out.l1.b'][None, :],
        p['out.l2.W'].T, p['out.l2.b'][None, :],
        p['out.l3.W'], p['out.l3.b'][None, :],
        p['out.prelu1'][None, :], p['out.prelu2'][None, :],
    )
    return out[:, 0]


# speedup vs baseline: 23.3037x; 1.2504x over previous
import functools
import math

import jax
import jax.numpy as jnp
from jax.experimental import pallas as pl
from jax.experimental.pallas import tpu as pltpu

_NEG = -1e30


def _b16(x):
    return x.astype(jnp.bfloat16).astype(jnp.float32)


def _bdot(a, b):
    return jnp.dot(a.astype(jnp.bfloat16), b.astype(jnp.bfloat16),
                   preferred_element_type=jnp.float32)


def _pick_rows(nres: int) -> int:
    best = 0
    for r in range(1, min(nres, 2048) + 1):
        if nres % r == 0 and (r % 8 == 0 or best == 0):
            best = r
    return best if best else nres


def _att_kernel(nblk, layer, hid, pe_ref, seg_ref, w2t_ref, wkcat_ref,
                bkcat_ref, w1blk_ref, out_ref, m_ref, s_ref, o_ref):
    i = pl.program_id(0)
    bsz = s_ref.shape[0]

    @pl.when(i == 0)
    def _init():
        m_ref[...] = jnp.full(m_ref.shape, _NEG, jnp.float32)
        s_ref[...] = jnp.zeros(s_ref.shape, jnp.float32)
        o_ref[...] = jnp.zeros(o_ref.shape, jnp.float32)

    pe = pe_ref[...]
    prot16 = jnp.maximum(
        jnp.dot(pe.astype(jnp.bfloat16), w2t_ref[...],
                preferred_element_type=jnp.float32),
        0.0).astype(jnp.bfloat16)
    rows = prot16.shape[0]
    seg_row = seg_ref[0]
    onehot = (seg_row == jax.lax.broadcasted_iota(
        jnp.int32, (bsz, rows), 0)).astype(jnp.bfloat16)

    kall = jnp.maximum(
        jnp.dot(prot16, wkcat_ref[...], preferred_element_type=jnp.float32)
        .astype(jnp.bfloat16) + bkcat_ref[...],
        jnp.bfloat16(0.0))
    t_all = jnp.dot(kall, w1blk_ref[...],
                    preferred_element_type=jnp.float32)
    m_new = jnp.maximum(m_ref[...], jnp.max(t_all, axis=0, keepdims=True))
    alpha = _b16(jnp.exp(m_ref[...] - m_new))
    e16 = jnp.exp(t_all - m_new).astype(jnp.bfloat16)
    s_ref[...] = alpha * s_ref[...] + jax.lax.dot_general(
        onehot, e16, (((1,), (0,)), ((), ())),
        preferred_element_type=jnp.float32)
    e_wide = jnp.repeat(e16, hid, axis=1)
    a_wide = jnp.repeat(alpha, hid, axis=1)
    o_ref[...] = a_wide * o_ref[...] + jax.lax.dot_general(
        onehot, kall * e_wide, (((1,), (0,)), ((), ())),
        preferred_element_type=jnp.float32)
    m_ref[...] = m_new

    @pl.when(i == nblk - 1)
    def _fin():
        for l in range(layer):
            out_ref[l] = (o_ref[:, l * hid:(l + 1) * hid]
                          / (s_ref[:, l:l + 1] + 1e-16))


def _ln(x, g, b, eps=1e-5):
    m = jnp.mean(x, axis=-1, keepdims=True)
    xc = x - m
    v = jnp.mean(xc * xc, axis=-1, keepdims=True)
    return xc * jax.lax.rsqrt(v + eps) * g + b


def _gelu(x):
    return 0.5 * x * (1.0 + jax.lax.erf(x / math.sqrt(2.0)))


def _softmax_pool(q, keys, vals, hid):
    scale = 1.0 / math.sqrt(hid)
    q16 = _b16(q)
    att = [jnp.sum(q16 * _b16(kk), axis=1, keepdims=True) * scale
           for kk in keys]
    mx = att[0]
    for a in att[1:]:
        mx = jnp.maximum(mx, a)
    es = [jnp.exp(a - mx) for a in att]
    den = es[0]
    for ee in es[1:]:
        den = den + ee
    ws = [ee / den for ee in es]
    acc = _b16(ws[0]) * _b16(vals[0])
    for ww, vv in zip(ws[1:], vals[1:]):
        acc = acc + _b16(ww) * _b16(vv)
    return acc


def _head_kernel(npatch, layer, hid,
                 x_ref, ap_ref, fpg_ref, fpb_ref, fpw_ref, fpbias_ref,
                 gdg_ref, gdb_ref, gd1w_ref, gd1b_ref, gd2w_ref, gd2b_ref,
                 gpw_ref, gpb_ref, qw_ref, qb_ref, lndg_ref, lndb_ref,
                 kdw_ref, kdb_ref, lnpg_ref, lnpb_ref, kpw_ref, kpb_ref,
                 l1w_ref, l1b_ref, l2w_ref, l2b_ref, l3w_ref, l3b_ref,
                 pr1_ref, pr2_ref, out_ref):
    x = x_ref[...]
    dtok = _bdot(_ln(x, fpg_ref[...], fpb_ref[...]),
                 fpw_ref[...]) + fpbias_ref[...]
    d_toks, kd = [], []
    for n in range(npatch):
        dn = _ln(dtok[:, n * hid:(n + 1) * hid], lndg_ref[...], lndb_ref[...])
        d_toks.append(dn)
        kd.append(_bdot(dn, kdw_ref[...]) + kdb_ref[...])
    g = _bdot(_ln(x, gdg_ref[...], gdb_ref[...]), gd1w_ref[...]) + gd1b_ref[...]
    g_d = _bdot(_gelu(g), gd2w_ref[...]) + gd2b_ref[...]
    ap_mean = ap_ref[0]
    for l in range(1, layer):
        ap_mean = ap_mean + ap_ref[l]
    ap_mean = ap_mean * (1.0 / layer)
    g_p = _gelu(_bdot(ap_mean, gpw_ref[...]) + gpb_ref[...])
    q_all = (_bdot(g_d, qw_ref[:hid]) + _bdot(g_p, qw_ref[hid:])
             + qb_ref[...])
    pts, kp = [], []
    for l in range(layer):
        pt = _ln(ap_ref[l], lnpg_ref[...], lnpb_ref[...])
        pts.append(pt)
        kp.append(_bdot(pt, kpw_ref[...]) + kpb_ref[...])
    vp = _softmax_pool(q_all, kp, pts, hid)
    vd = _softmax_pool(q_all, kd, d_toks, hid)
    z = _bdot(vd, l1w_ref[:hid]) + _bdot(vp, l1w_ref[hid:]) + l1b_ref[...]
    z = jnp.where(z >= 0, z, pr1_ref[...] * z)
    z = _bdot(z, l2w_ref[...]) + l2b_ref[...]
    z = jnp.where(z >= 0, z, pr2_ref[...] * z)
    out_ref[...] = jnp.sum(_b16(z) * _b16(l3w_ref[...]), axis=1,
                           keepdims=True) + l3b_ref[...]


def kernel(x, pro_emb, params, pro_emb_batch):
    p = params
    bsz, _ = x.shape
    nres, pdim = pro_emb.shape
    hid = p['prej2.W'].shape[0]
    layer = len([k for k in p if k.startswith('att') and k.endswith('.q.W')])
    npatch = p['ccfm.fp_proj.W'].shape[0] // hid
    rows = _pick_rows(nres)
    nblk = nres // rows

    w2t = p['prej2.W'].T.astype(jnp.bfloat16)
    wkcat = jnp.concatenate([p['att%d.k.W' % l].T for l in range(layer)],
                            axis=1).astype(jnp.bfloat16)
    bkcat = jnp.concatenate([p['att%d.k.b' % l] for l in range(layer)]
                            )[None, :].astype(jnp.bfloat16)
    w1cat = jnp.concatenate([p['att%d.merge.W' % l][0, :hid]
                             for l in range(layer)])
    lheye = jnp.repeat(jnp.eye(layer, dtype=jnp.float32), hid, axis=0)
    w1blk = (lheye * w1cat[:, None]).astype(jnp.bfloat16)
    seg3 = pro_emb_batch.reshape(nblk, 1, rows)

    att = pl.pallas_call(
        functools.partial(_att_kernel, nblk, layer, hid),
        grid=(nblk,),
        in_specs=[
            pl.BlockSpec((rows, pdim), lambda i: (i, 0)),
            pl.BlockSpec((1, 1, rows), lambda i: (i, 0, 0)),
            pl.BlockSpec((pdim, hid), lambda i: (0, 0)),
            pl.BlockSpec((hid, layer * hid), lambda i: (0, 0)),
            pl.BlockSpec((1, layer * hid), lambda i: (0, 0)),
            pl.BlockSpec((layer * hid, layer), lambda i: (0, 0)),
        ],
        out_specs=pl.BlockSpec((layer, bsz, hid), lambda i: (0, 0, 0)),
        out_shape=jax.ShapeDtypeStruct((layer, bsz, hid), jnp.float32),
        scratch_shapes=[
            pltpu.VMEM((1, layer), jnp.float32),
            pltpu.VMEM((bsz, layer), jnp.float32),
            pltpu.VMEM((bsz, layer * hid), jnp.float32),
        ],
        compiler_params=pltpu.CompilerParams(
            dimension_semantics=("arbitrary",)),
    )(pro_emb, seg3, w2t, wkcat, bkcat, w1blk)

    out = pl.pallas_call(
        functools.partial(_head_kernel, npatch, layer, hid),
        out_shape=jax.ShapeDtypeStruct((bsz, 1), jnp.float32),
    )(
        x, att,
        p['ccfm.fp_ln.g'][None, :], p['ccfm.fp_ln.b'][None, :],
        p['ccfm.fp_proj.W'].T, p['ccfm.fp_proj.b'][None, :],
        p['ccfm.gd_ln.g'][None, :], p['ccfm.gd_ln.b'][None, :],
        p['ccfm.gd1.W'].T, p['ccfm.gd1.b'][None, :],
        p['ccfm.gd2.W'].T, p['ccfm.gd2.b'][None, :],
        p['ccfm.gp.W'].T, p['ccfm.gp.b'][None, :],
        p['ccfm.q.W'].T, p['ccfm.q.b'][None, :],
        p['ccfm.ln_d.g'][None, :], p['ccfm.ln_d.b'][None, :],
        p['ccfm.k_d.W'].T, p['ccfm.k_d.b'][None, :],
        p['ccfm.ln_p.g'][None, :], p['ccfm.ln_p.b'][None, :],
        p['ccfm.k_p.W'].T, p['ccfm.k_p.b'][None, :],
        p['out.l1.W'].T, p['out.l1.b'][None, :],
        p['out.l2.W'].T, p['out.l2.b'][None, :],
        p['out.l3.W'], p['out.l3.b'][None, :],
        p['out.prelu1'][None, :], p['out.prelu2'][None, :],
    )
    return out[:, 0]


# speedup vs baseline: 23.3853x; 1.0035x over previous
import functools
import math

import jax
import jax.numpy as jnp
from jax.experimental import pallas as pl
from jax.experimental.pallas import tpu as pltpu

_NEG = -1e30


def _b16(x):
    return x.astype(jnp.bfloat16).astype(jnp.float32)


def _bdot(a, b):
    return jnp.dot(a.astype(jnp.bfloat16), b.astype(jnp.bfloat16),
                   preferred_element_type=jnp.float32)


def _bdot_t(a, b):
    return jax.lax.dot_general(
        a.astype(jnp.bfloat16), b.astype(jnp.bfloat16),
        (((1,), (1,)), ((), ())), preferred_element_type=jnp.float32)


def _pick_rows(nres: int) -> int:
    best = 0
    for r in range(1, min(nres, 2048) + 1):
        if nres % r == 0 and (r % 8 == 0 or best == 0):
            best = r
    return best if best else nres


def _ln(x, g, b, eps=1e-5):
    m = jnp.mean(x, axis=-1, keepdims=True)
    xc = x - m
    v = jnp.mean(xc * xc, axis=-1, keepdims=True)
    return xc * jax.lax.rsqrt(v + eps) * g + b


def _gelu(x):
    return 0.5 * x * (1.0 + jax.lax.erf(x / math.sqrt(2.0)))


def _softmax_pool(q, keys, vals, hid):
    scale = 1.0 / math.sqrt(hid)
    q16 = _b16(q)
    att = [jnp.sum(q16 * _b16(kk), axis=1, keepdims=True) * scale
           for kk in keys]
    mx = att[0]
    for a in att[1:]:
        mx = jnp.maximum(mx, a)
    es = [jnp.exp(a - mx) for a in att]
    den = es[0]
    for ee in es[1:]:
        den = den + ee
    ws = [ee / den for ee in es]
    acc = _b16(ws[0]) * _b16(vals[0])
    for ww, vv in zip(ws[1:], vals[1:]):
        acc = acc + _b16(ww) * _b16(vv)
    return acc


def _head(npatch, layer, hid, x, ap,
          fpg_ref, fpb_ref, fpw_ref, fpbias_ref,
          gdg_ref, gdb_ref, gd1w_ref, gd1b_ref, gd2w_ref, gd2b_ref,
          gpw_ref, gpb_ref, qw_ref, qb_ref, lndg_ref, lndb_ref,
          kdw_ref, kdb_ref, lnpg_ref, lnpb_ref, kpw_ref, kpb_ref,
          l1w_ref, l1b_ref, l2w_ref, l2b_ref, l3w_ref, l3b_ref,
          pr1_ref, pr2_ref, out_ref):
    dtok = _bdot_t(_ln(x, fpg_ref[...], fpb_ref[...]),
                   fpw_ref[...]) + fpbias_ref[...]
    d_toks, kd = [], []
    for n in range(npatch):
        dn = _ln(dtok[:, n * hid:(n + 1) * hid], lndg_ref[...], lndb_ref[...])
        d_toks.append(dn)
        kd.append(_bdot_t(dn, kdw_ref[...]) + kdb_ref[...])
    g = _bdot_t(_ln(x, gdg_ref[...], gdb_ref[...]),
                gd1w_ref[...]) + gd1b_ref[...]
    g_d = _bdot_t(_gelu(g), gd2w_ref[...]) + gd2b_ref[...]
    ap_mean = ap[0]
    for l in range(1, layer):
        ap_mean = ap_mean + ap[l]
    ap_mean = ap_mean * (1.0 / layer)
    g_p = _gelu(_bdot_t(ap_mean, gpw_ref[...]) + gpb_ref[...])
    q_all = (_bdot_t(g_d, qw_ref[:, :hid]) + _bdot_t(g_p, qw_ref[:, hid:])
             + qb_ref[...])
    pts, kp = [], []
    for l in range(layer):
        pt = _ln(ap[l], lnpg_ref[...], lnpb_ref[...])
        pts.append(pt)
        kp.append(_bdot_t(pt, kpw_ref[...]) + kpb_ref[...])
    vp = _softmax_pool(q_all, kp, pts, hid)
    vd = _softmax_pool(q_all, kd, d_toks, hid)
    z = (_bdot_t(vd, l1w_ref[:, :hid]) + _bdot_t(vp, l1w_ref[:, hid:])
         + l1b_ref[...])
    z = jnp.where(z >= 0, z, pr1_ref[...] * z)
    z = _bdot_t(z, l2w_ref[...]) + l2b_ref[...]
    z = jnp.where(z >= 0, z, pr2_ref[...] * z)
    out_ref[...] = jnp.sum(_b16(z) * _b16(l3w_ref[...]), axis=1,
                           keepdims=True) + l3b_ref[...]


def _att_kernel(nblk, layer, hid, npatch,
                pe_ref, seg_ref, w2t_ref, wkcat_ref, bkcat_ref, w1blk_ref,
                x_ref, *head_refs):
    out_ref, m_ref, s_ref, o_ref = head_refs[-4:]
    i = pl.program_id(0)
    bsz = s_ref.shape[0]

    @pl.when(i == 0)
    def _init():
        m_ref[...] = jnp.full(m_ref.shape, _NEG, jnp.float32)
        s_ref[...] = jnp.zeros(s_ref.shape, jnp.float32)
        o_ref[...] = jnp.zeros(o_ref.shape, jnp.float32)

    pe = pe_ref[...]
    prot16 = jnp.maximum(
        jnp.dot(pe.astype(jnp.bfloat16), w2t_ref[...],
                preferred_element_type=jnp.float32),
        0.0).astype(jnp.bfloat16)
    rows = prot16.shape[0]
    seg_row = seg_ref[0]
    onehot = (seg_row == jax.lax.broadcasted_iota(
        jnp.int32, (bsz, rows), 0)).astype(jnp.bfloat16)

    kall = jnp.maximum(
        jnp.dot(prot16, wkcat_ref[...], preferred_element_type=jnp.float32)
        .astype(jnp.bfloat16) + bkcat_ref[...],
        jnp.bfloat16(0.0))
    t_all = jnp.dot(kall, w1blk_ref[...],
                    preferred_element_type=jnp.float32)
    m_new = jnp.maximum(m_ref[...], jnp.max(t_all, axis=0, keepdims=True))
    alpha = _b16(jnp.exp(m_ref[...] - m_new))
    e16 = jnp.exp(t_all - m_new).astype(jnp.bfloat16)
    s_ref[...] = alpha * s_ref[...] + jax.lax.dot_general(
        onehot, e16, (((1,), (0,)), ((), ())),
        preferred_element_type=jnp.float32)
    e_wide = jnp.repeat(e16, hid, axis=1)
    a_wide = jnp.repeat(alpha, hid, axis=1)
    o_ref[...] = a_wide * o_ref[...] + jax.lax.dot_general(
        onehot, kall * e_wide, (((1,), (0,)), ((), ())),
        preferred_element_type=jnp.float32)
    m_ref[...] = m_new

    @pl.when(i == nblk - 1)
    def _fin():
        ap = [o_ref[:, l * hid:(l + 1) * hid]
              / (s_ref[:, l:l + 1] + 1e-16) for l in range(layer)]
        _head(npatch, layer, hid, x_ref[...], ap, *head_refs[:-3])


def kernel(x, pro_emb, params, pro_emb_batch):
    p = params
    bsz, mol_in = x.shape
    nres, pdim = pro_emb.shape
    hid = p['prej2.W'].shape[0]
    layer = len([k for k in p if k.startswith('att') and k.endswith('.q.W')])
    npatch = p['ccfm.fp_proj.W'].shape[0] // hid
    rows = _pick_rows(nres)
    nblk = nres // rows

    w2t = p['prej2.W'].T.astype(jnp.bfloat16)
    wkcat = jnp.concatenate([p['att%d.k.W' % l].T for l in range(layer)],
                            axis=1).astype(jnp.bfloat16)
    bkcat = jnp.concatenate([p['att%d.k.b' % l] for l in range(layer)]
                            )[None, :].astype(jnp.bfloat16)
    w1cat = jnp.concatenate([p['att%d.merge.W' % l][0, :hid]
                             for l in range(layer)])
    lheye = jnp.repeat(jnp.eye(layer, dtype=jnp.float32), hid, axis=0)
    w1blk = (lheye * w1cat[:, None]).astype(jnp.bfloat16)
    seg3 = pro_emb_batch.reshape(nblk, 1, rows)

    head_params = [
        p['ccfm.fp_ln.g'][None, :], p['ccfm.fp_ln.b'][None, :],
        p['ccfm.fp_proj.W'], p['ccfm.fp_proj.b'][None, :],
        p['ccfm.gd_ln.g'][None, :], p['ccfm.gd_ln.b'][None, :],
        p['ccfm.gd1.W'], p['ccfm.gd1.b'][None, :],
        p['ccfm.gd2.W'], p['ccfm.gd2.b'][None, :],
        p['ccfm.gp.W'], p['ccfm.gp.b'][None, :],
        p['ccfm.q.W'], p['ccfm.q.b'][None, :],
        p['ccfm.ln_d.g'][None, :], p['ccfm.ln_d.b'][None, :],
        p['ccfm.k_d.W'], p['ccfm.k_d.b'][None, :],
        p['ccfm.ln_p.g'][None, :], p['ccfm.ln_p.b'][None, :],
        p['ccfm.k_p.W'], p['ccfm.k_p.b'][None, :],
        p['out.l1.W'], p['out.l1.b'][None, :],
        p['out.l2.W'], p['out.l2.b'][None, :],
        p['out.l3.W'], p['out.l3.b'][None, :],
        p['out.prelu1'][None, :], p['out.prelu2'][None, :],
    ]

    def _const2(shape):
        return pl.BlockSpec(shape, lambda i: (0, 0))

    out = pl.pallas_call(
        functools.partial(_att_kernel, nblk, layer, hid, npatch),
        grid=(nblk,),
        in_specs=[
            pl.BlockSpec((rows, pdim), lambda i: (i, 0)),
            pl.BlockSpec((1, 1, rows), lambda i: (i, 0, 0)),
            _const2((pdim, hid)),
            _const2((hid, layer * hid)),
            _const2((1, layer * hid)),
            _const2((layer * hid, layer)),
            _const2((bsz, mol_in)),
        ] + [_const2(hp.shape) for hp in head_params],
        out_specs=pl.BlockSpec((bsz, 1), lambda i: (0, 0)),
        out_shape=jax.ShapeDtypeStruct((bsz, 1), jnp.float32),
        scratch_shapes=[
            pltpu.VMEM((1, layer), jnp.float32),
            pltpu.VMEM((bsz, layer), jnp.float32),
            pltpu.VMEM((bsz, layer * hid), jnp.float32),
        ],
        compiler_params=pltpu.CompilerParams(
            dimension_semantics=("arbitrary",)),
    )(pro_emb, seg3, w2t, wkcat, bkcat, w1blk, x, *head_params)
    return out[:, 0]


# speedup vs baseline: 25.9183x; 1.1083x over previous
import functools
import math

import jax
import jax.numpy as jnp
from jax.experimental import pallas as pl
from jax.experimental.pallas import tpu as pltpu

_NEG = -1e30


def _b16(x):
    return x.astype(jnp.bfloat16).astype(jnp.float32)


def _bdot(a, b):
    return jnp.dot(a.astype(jnp.bfloat16), b.astype(jnp.bfloat16),
                   preferred_element_type=jnp.float32)


def _bdot_t(a, b):
    return jax.lax.dot_general(
        a.astype(jnp.bfloat16), b.astype(jnp.bfloat16),
        (((1,), (1,)), ((), ())), preferred_element_type=jnp.float32)


def _pick_rows(nres: int) -> int:
    best = 0
    for r in range(1, min(nres, 2048) + 1):
        if nres % r == 0 and (r % 8 == 0 or best == 0):
            best = r
    return best if best else nres


def _ln(x, g, b, eps=1e-5):
    m = jnp.mean(x, axis=-1, keepdims=True)
    xc = x - m
    v = jnp.mean(xc * xc, axis=-1, keepdims=True)
    return xc * jax.lax.rsqrt(v + eps) * g + b


def _gelu(x):
    return 0.5 * x * (1.0 + jax.lax.erf(x / math.sqrt(2.0)))


def _softmax_pool(q, keys, vals, hid):
    scale = 1.0 / math.sqrt(hid)
    q16 = _b16(q)
    att = [jnp.sum(q16 * _b16(kk), axis=1, keepdims=True) * scale
           for kk in keys]
    mx = att[0]
    for a in att[1:]:
        mx = jnp.maximum(mx, a)
    es = [jnp.exp(a - mx) for a in att]
    den = es[0]
    for ee in es[1:]:
        den = den + ee
    ws = [ee / den for ee in es]
    acc = _b16(ws[0]) * _b16(vals[0])
    for ww, vv in zip(ws[1:], vals[1:]):
        acc = acc + _b16(ww) * _b16(vv)
    return acc


def _head(npatch, layer, hid, x, ap,
          fpg_ref, fpb_ref, fpw_ref, fpbias_ref,
          gdg_ref, gdb_ref, gd1w_ref, gd1b_ref, gd2w_ref, gd2b_ref,
          gpw_ref, gpb_ref, qw_ref, qb_ref, lndg_ref, lndb_ref,
          kdw_ref, kdb_ref, lnpg_ref, lnpb_ref, kpw_ref, kpb_ref,
          l1w_ref, l1b_ref, l2w_ref, l2b_ref, l3w_ref, l3b_ref,
          pr1_ref, pr2_ref, out_ref):
    dtok = _bdot_t(_ln(x, fpg_ref[...], fpb_ref[...]),
                   fpw_ref[...]) + fpbias_ref[...]
    d_toks, kd = [], []
    for n in range(npatch):
        dn = _ln(dtok[:, n * hid:(n + 1) * hid], lndg_ref[...], lndb_ref[...])
        d_toks.append(dn)
        kd.append(_bdot_t(dn, kdw_ref[...]) + kdb_ref[...])
    g = _bdot_t(_ln(x, gdg_ref[...], gdb_ref[...]),
                gd1w_ref[...]) + gd1b_ref[...]
    g_d = _bdot_t(_gelu(g), gd2w_ref[...]) + gd2b_ref[...]
    ap_mean = ap[0]
    for l in range(1, layer):
        ap_mean = ap_mean + ap[l]
    ap_mean = ap_mean * (1.0 / layer)
    g_p = _gelu(_bdot_t(ap_mean, gpw_ref[...]) + gpb_ref[...])
    q_all = (_bdot_t(g_d, qw_ref[:, :hid]) + _bdot_t(g_p, qw_ref[:, hid:])
             + qb_ref[...])
    pts, kp = [], []
    for l in range(layer):
        pt = _ln(ap[l], lnpg_ref[...], lnpb_ref[...])
        pts.append(pt)
        kp.append(_bdot_t(pt, kpw_ref[...]) + kpb_ref[...])
    vp = _softmax_pool(q_all, kp, pts, hid)
    vd = _softmax_pool(q_all, kd, d_toks, hid)
    z = (_bdot_t(vd, l1w_ref[:, :hid]) + _bdot_t(vp, l1w_ref[:, hid:])
         + l1b_ref[...])
    z = jnp.where(z >= 0, z, pr1_ref[...] * z)
    z = _bdot_t(z, l2w_ref[...]) + l2b_ref[...]
    z = jnp.where(z >= 0, z, pr2_ref[...] * z)
    out_ref[...] = jnp.sum(_b16(z) * _b16(l3w_ref[...]), axis=1,
                           keepdims=True) + l3b_ref[...]


def _att_kernel(nblk, layer, hid, npatch,
                pe_ref, seg_ref, w2t_ref, wkcat_ref, bkcat_ref, w1blk_ref,
                x_ref, *head_refs):
    out_ref, m_ref, s_ref, o_ref = head_refs[-4:]
    i = pl.program_id(0)
    bsz = s_ref.shape[0]

    @pl.when(i == 0)
    def _init():
        m_ref[...] = jnp.full(m_ref.shape, _NEG, jnp.float32)
        s_ref[...] = jnp.zeros(s_ref.shape, jnp.float32)
        o_ref[...] = jnp.zeros(o_ref.shape, jnp.float32)

    pe = pe_ref[...]
    prot16 = jnp.maximum(
        jnp.dot(pe.astype(jnp.bfloat16), w2t_ref[...],
                preferred_element_type=jnp.float32),
        0.0).astype(jnp.bfloat16)
    rows = prot16.shape[0]
    seg_row = seg_ref[0]

    kall = jnp.maximum(
        jnp.dot(prot16, wkcat_ref[...], preferred_element_type=jnp.float32)
        .astype(jnp.bfloat16) + bkcat_ref[...],
        jnp.bfloat16(0.0))
    t_all = jnp.dot(kall, w1blk_ref[...],
                    preferred_element_type=jnp.float32)
    m_old = m_ref[...]
    m_new = jnp.maximum(m_old, jnp.max(t_all, axis=0, keepdims=True))
    alpha = _b16(jnp.exp(m_old - m_new))
    e16 = jnp.exp(t_all - m_new).astype(jnp.bfloat16)
    e_wide = jnp.repeat(e16, hid, axis=1)
    ke = kall * e_wide

    @pl.when(jnp.any(m_new > m_old))
    def _rescale():
        s_ref[...] = alpha * s_ref[...]
        o_ref[...] = jnp.repeat(alpha, hid, axis=1) * o_ref[...]
    m_ref[...] = m_new

    lo = jnp.min(seg_row)
    hi = jnp.max(seg_row)
    sb = 32 if bsz % 32 == 0 else bsz
    for j in range(bsz // sb):
        base = j * sb

        @pl.when((hi >= base) & (lo < base + sb))
        def _band(base=base):
            oh = (seg_row - base == jax.lax.broadcasted_iota(
                jnp.int32, (sb, rows), 0)).astype(jnp.bfloat16)
            s_ref[base:base + sb, :] = (
                s_ref[base:base + sb, :] + jax.lax.dot_general(
                    oh, e16, (((1,), (0,)), ((), ())),
                    preferred_element_type=jnp.float32))
            o_ref[base:base + sb, :] = (
                o_ref[base:base + sb, :] + jax.lax.dot_general(
                    oh, ke, (((1,), (0,)), ((), ())),
                    preferred_element_type=jnp.float32))

    @pl.when(i == nblk - 1)
    def _fin():
        ap = [o_ref[:, l * hid:(l + 1) * hid]
              / (s_ref[:, l:l + 1] + 1e-16) for l in range(layer)]
        _head(npatch, layer, hid, x_ref[...], ap, *head_refs[:-3])


def kernel(x, pro_emb, params, pro_emb_batch):
    p = params
    bsz, mol_in = x.shape
    nres, pdim = pro_emb.shape
    hid = p['prej2.W'].shape[0]
    layer = len([k for k in p if k.startswith('att') and k.endswith('.q.W')])
    npatch = p['ccfm.fp_proj.W'].shape[0] // hid
    rows = _pick_rows(nres)
    nblk = nres // rows

    w2t = p['prej2.W'].T.astype(jnp.bfloat16)
    wkcat = jnp.concatenate([p['att%d.k.W' % l].T for l in range(layer)],
                            axis=1).astype(jnp.bfloat16)
    bkcat = jnp.concatenate([p['att%d.k.b' % l] for l in range(layer)]
                            )[None, :].astype(jnp.bfloat16)
    w1cat = jnp.concatenate([p['att%d.merge.W' % l][0, :hid]
                             for l in range(layer)])
    lheye = jnp.repeat(jnp.eye(layer, dtype=jnp.float32), hid, axis=0)
    w1blk = (lheye * w1cat[:, None]).astype(jnp.bfloat16)
    seg3 = pro_emb_batch.reshape(nblk, 1, rows)

    head_params = [
        p['ccfm.fp_ln.g'][None, :], p['ccfm.fp_ln.b'][None, :],
        p['ccfm.fp_proj.W'], p['ccfm.fp_proj.b'][None, :],
        p['ccfm.gd_ln.g'][None, :], p['ccfm.gd_ln.b'][None, :],
        p['ccfm.gd1.W'], p['ccfm.gd1.b'][None, :],
        p['ccfm.gd2.W'], p['ccfm.gd2.b'][None, :],
        p['ccfm.gp.W'], p['ccfm.gp.b'][None, :],
        p['ccfm.q.W'], p['ccfm.q.b'][None, :],
        p['ccfm.ln_d.g'][None, :], p['ccfm.ln_d.b'][None, :],
        p['ccfm.k_d.W'], p['ccfm.k_d.b'][None, :],
        p['ccfm.ln_p.g'][None, :], p['ccfm.ln_p.b'][None, :],
        p['ccfm.k_p.W'], p['ccfm.k_p.b'][None, :],
        p['out.l1.W'], p['out.l1.b'][None, :],
        p['out.l2.W'], p['out.l2.b'][None, :],
        p['out.l3.W'], p['out.l3.b'][None, :],
        p['out.prelu1'][None, :], p['out.prelu2'][None, :],
    ]

    def _const2(shape):
        return pl.BlockSpec(shape, lambda i: (0, 0))

    out = pl.pallas_call(
        functools.partial(_att_kernel, nblk, layer, hid, npatch),
        grid=(nblk,),
        in_specs=[
            pl.BlockSpec((rows, pdim), lambda i: (i, 0)),
            pl.BlockSpec((1, 1, rows), lambda i: (i, 0, 0)),
            _const2((pdim, hid)),
            _const2((hid, layer * hid)),
            _const2((1, layer * hid)),
            _const2((layer * hid, layer)),
            _const2((bsz, mol_in)),
        ] + [_const2(hp.shape) for hp in head_params],
        out_specs=pl.BlockSpec((bsz, 1), lambda i: (0, 0)),
        out_shape=jax.ShapeDtypeStruct((bsz, 1), jnp.float32),
        scratch_shapes=[
            pltpu.VMEM((1, layer), jnp.float32),
            pltpu.VMEM((bsz, layer), jnp.float32),
            pltpu.VMEM((bsz, layer * hid), jnp.float32),
        ],
        compiler_params=pltpu.CompilerParams(
            dimension_semantics=("arbitrary",)),
    )(pro_emb, seg3, w2t, wkcat, bkcat, w1blk, x, *head_params)
    return out[:, 0]


# speedup vs baseline: 31.2791x; 1.2068x over previous
import functools
import math

import jax
import jax.numpy as jnp
from jax.experimental import pallas as pl
from jax.experimental.pallas import tpu as pltpu

_NEG = -1e30


def _b16(x):
    return x.astype(jnp.bfloat16).astype(jnp.float32)


def _bdot(a, b):
    return jnp.dot(a.astype(jnp.bfloat16), b.astype(jnp.bfloat16),
                   preferred_element_type=jnp.float32)


def _bdot_t(a, b):
    return jax.lax.dot_general(
        a.astype(jnp.bfloat16), b.astype(jnp.bfloat16),
        (((1,), (1,)), ((), ())), preferred_element_type=jnp.float32)


def _pick_rows(nres: int) -> int:
    best = 0
    for r in range(1, min(nres, 2048) + 1):
        if nres % r == 0 and (r % 8 == 0 or best == 0):
            best = r
    return best if best else nres


def _ln(x, g, b, eps=1e-5):
    m = jnp.mean(x, axis=-1, keepdims=True)
    xc = x - m
    v = jnp.mean(xc * xc, axis=-1, keepdims=True)
    return xc * jax.lax.rsqrt(v + eps) * g + b


def _gelu(x):
    return 0.5 * x * (1.0 + jax.lax.erf(x / math.sqrt(2.0)))


def _softmax_pool(q, keys, vals, hid):
    scale = 1.0 / math.sqrt(hid)
    q16 = _b16(q)
    att = [jnp.sum(q16 * _b16(kk), axis=1, keepdims=True) * scale
           for kk in keys]
    mx = att[0]
    for a in att[1:]:
        mx = jnp.maximum(mx, a)
    es = [jnp.exp(a - mx) for a in att]
    den = es[0]
    for ee in es[1:]:
        den = den + ee
    ws = [ee / den for ee in es]
    acc = _b16(ws[0]) * _b16(vals[0])
    for ww, vv in zip(ws[1:], vals[1:]):
        acc = acc + _b16(ww) * _b16(vv)
    return acc


def _head(npatch, layer, hid, x, ap,
          fpg_ref, fpb_ref, fpw_ref, fpbias_ref,
          gdg_ref, gdb_ref, gd1w_ref, gd1b_ref, gd2w_ref, gd2b_ref,
          gpw_ref, gpb_ref, qw_ref, qb_ref, lndg_ref, lndb_ref,
          kdw_ref, kdb_ref, lnpg_ref, lnpb_ref, kpw_ref, kpb_ref,
          l1w_ref, l1b_ref, l2w_ref, l2b_ref, l3w_ref, l3b_ref,
          pr1_ref, pr2_ref, out_ref):
    dtok = _bdot_t(_ln(x, fpg_ref[...], fpb_ref[...]),
                   fpw_ref[...]) + fpbias_ref[...]
    d_toks, kd = [], []
    for n in range(npatch):
        dn = _ln(dtok[:, n * hid:(n + 1) * hid], lndg_ref[...], lndb_ref[...])
        d_toks.append(dn)
        kd.append(_bdot_t(dn, kdw_ref[...]) + kdb_ref[...])
    g = _bdot_t(_ln(x, gdg_ref[...], gdb_ref[...]),
                gd1w_ref[...]) + gd1b_ref[...]
    g_d = _bdot_t(_gelu(g), gd2w_ref[...]) + gd2b_ref[...]
    ap_mean = ap[0]
    for l in range(1, layer):
        ap_mean = ap_mean + ap[l]
    ap_mean = ap_mean * (1.0 / layer)
    g_p = _gelu(_bdot_t(ap_mean, gpw_ref[...]) + gpb_ref[...])
    q_all = (_bdot_t(g_d, qw_ref[:, :hid]) + _bdot_t(g_p, qw_ref[:, hid:])
             + qb_ref[...])
    pts, kp = [], []
    for l in range(layer):
        pt = _ln(ap[l], lnpg_ref[...], lnpb_ref[...])
        pts.append(pt)
        kp.append(_bdot_t(pt, kpw_ref[...]) + kpb_ref[...])
    vp = _softmax_pool(q_all, kp, pts, hid)
    vd = _softmax_pool(q_all, kd, d_toks, hid)
    z = (_bdot_t(vd, l1w_ref[:, :hid]) + _bdot_t(vp, l1w_ref[:, hid:])
         + l1b_ref[...])
    z = jnp.where(z >= 0, z, pr1_ref[...] * z)
    z = _bdot_t(z, l2w_ref[...]) + l2b_ref[...]
    z = jnp.where(z >= 0, z, pr2_ref[...] * z)
    out_ref[...] = jnp.sum(_b16(z) * _b16(l3w_ref[...]), axis=1,
                           keepdims=True) + l3b_ref[...]


def _att_kernel(nblk, layer, hid, npatch,
                pe_ref, seg_ref, w2t_ref, wkcat_ref, bkcat_ref, w1blk_ref,
                x_ref, *head_refs):
    out_ref, m_ref, s_ref, o_ref = head_refs[-4:]
    i = pl.program_id(0)
    bsz = s_ref.shape[0]

    @pl.when(i == 0)
    def _init():
        m_ref[...] = jnp.full(m_ref.shape, _NEG, jnp.float32)
        s_ref[...] = jnp.zeros(s_ref.shape, jnp.float32)
        o_ref[...] = jnp.zeros(o_ref.shape, jnp.float32)

    pe = pe_ref[...]
    prot16 = jnp.maximum(
        jnp.dot(pe.astype(jnp.bfloat16), w2t_ref[...],
                preferred_element_type=jnp.float32),
        0.0).astype(jnp.bfloat16)
    rows = prot16.shape[0]
    seg_row = seg_ref[0]

    kall = jnp.maximum(
        jnp.dot(prot16, wkcat_ref[...], preferred_element_type=jnp.float32)
        .astype(jnp.bfloat16) + bkcat_ref[...],
        jnp.bfloat16(0.0))
    t_all = jnp.dot(kall, w1blk_ref[...],
                    preferred_element_type=jnp.float32)
    m_old = m_ref[...]
    m_new = jnp.maximum(m_old, jnp.max(t_all, axis=0, keepdims=True))
    alpha = _b16(jnp.exp(m_old - m_new))
    e16 = jnp.exp(t_all - m_new).astype(jnp.bfloat16)
    e16t = jnp.transpose(e16)

    @pl.when(jnp.any(m_new > m_old))
    def _rescale():
        s_ref[...] = alpha * s_ref[...]
        o_ref[...] = jnp.repeat(alpha, hid, axis=1) * o_ref[...]
    m_ref[...] = m_new

    lo = jnp.min(seg_row)
    hi = jnp.max(seg_row)
    sb = 32 if bsz % 32 == 0 else bsz
    for j in range(bsz // sb):
        base = j * sb

        @pl.when((hi >= base) & (lo < base + sb))
        def _band(base=base):
            oh = (seg_row - base == jax.lax.broadcasted_iota(
                jnp.int32, (sb, rows), 0)).astype(jnp.bfloat16)
            s_ref[base:base + sb, :] = (
                s_ref[base:base + sb, :] + jax.lax.dot_general(
                    oh, e16, (((1,), (0,)), ((), ())),
                    preferred_element_type=jnp.float32))
            o_band = jnp.concatenate(
                [jax.lax.dot_general(
                    oh * e16t[l:l + 1, :], kall[:, l * hid:(l + 1) * hid],
                    (((1,), (0,)), ((), ())),
                    preferred_element_type=jnp.float32)
                 for l in range(layer)], axis=1)
            o_ref[base:base + sb, :] = o_ref[base:base + sb, :] + o_band

    @pl.when(i == nblk - 1)
    def _fin():
        ap = [o_ref[:, l * hid:(l + 1) * hid]
              / (s_ref[:, l:l + 1] + 1e-16) for l in range(layer)]
        _head(npatch, layer, hid, x_ref[...], ap, *head_refs[:-3])


def kernel(x, pro_emb, params, pro_emb_batch):
    p = params
    bsz, mol_in = x.shape
    nres, pdim = pro_emb.shape
    hid = p['prej2.W'].shape[0]
    layer = len([k for k in p if k.startswith('att') and k.endswith('.q.W')])
    npatch = p['ccfm.fp_proj.W'].shape[0] // hid
    rows = _pick_rows(nres)
    nblk = nres // rows

    w2t = p['prej2.W'].T.astype(jnp.bfloat16)
    wkcat = jnp.concatenate([p['att%d.k.W' % l].T for l in range(layer)],
                            axis=1).astype(jnp.bfloat16)
    bkcat = jnp.concatenate([p['att%d.k.b' % l] for l in range(layer)]
                            )[None, :].astype(jnp.bfloat16)
    w1cat = jnp.concatenate([p['att%d.merge.W' % l][0, :hid]
                             for l in range(layer)])
    lheye = jnp.repeat(jnp.eye(layer, dtype=jnp.float32), hid, axis=0)
    w1blk = (lheye * w1cat[:, None]).astype(jnp.bfloat16)
    seg3 = pro_emb_batch.reshape(nblk, 1, rows)

    head_params = [
        p['ccfm.fp_ln.g'][None, :], p['ccfm.fp_ln.b'][None, :],
        p['ccfm.fp_proj.W'], p['ccfm.fp_proj.b'][None, :],
        p['ccfm.gd_ln.g'][None, :], p['ccfm.gd_ln.b'][None, :],
        p['ccfm.gd1.W'], p['ccfm.gd1.b'][None, :],
        p['ccfm.gd2.W'], p['ccfm.gd2.b'][None, :],
        p['ccfm.gp.W'], p['ccfm.gp.b'][None, :],
        p['ccfm.q.W'], p['ccfm.q.b'][None, :],
        p['ccfm.ln_d.g'][None, :], p['ccfm.ln_d.b'][None, :],
        p['ccfm.k_d.W'], p['ccfm.k_d.b'][None, :],
        p['ccfm.ln_p.g'][None, :], p['ccfm.ln_p.b'][None, :],
        p['ccfm.k_p.W'], p['ccfm.k_p.b'][None, :],
        p['out.l1.W'], p['out.l1.b'][None, :],
        p['out.l2.W'], p['out.l2.b'][None, :],
        p['out.l3.W'], p['out.l3.b'][None, :],
        p['out.prelu1'][None, :], p['out.prelu2'][None, :],
    ]

    def _const2(shape):
        return pl.BlockSpec(shape, lambda i: (0, 0))

    out = pl.pallas_call(
        functools.partial(_att_kernel, nblk, layer, hid, npatch),
        grid=(nblk,),
        in_specs=[
            pl.BlockSpec((rows, pdim), lambda i: (i, 0)),
            pl.BlockSpec((1, 1, rows), lambda i: (i, 0, 0)),
            _const2((pdim, hid)),
            _const2((hid, layer * hid)),
            _const2((1, layer * hid)),
            _const2((layer * hid, layer)),
            _const2((bsz, mol_in)),
        ] + [_const2(hp.shape) for hp in head_params],
        out_specs=pl.BlockSpec((bsz, 1), lambda i: (0, 0)),
        out_shape=jax.ShapeDtypeStruct((bsz, 1), jnp.float32),
        scratch_shapes=[
            pltpu.VMEM((1, layer), jnp.float32),
            pltpu.VMEM((bsz, layer), jnp.float32),
            pltpu.VMEM((bsz, layer * hid), jnp.float32),
        ],
        compiler_params=pltpu.CompilerParams(
            dimension_semantics=("arbitrary",)),
    )(pro_emb, seg3, w2t, wkcat, bkcat, w1blk, x, *head_params)
    return out[:, 0]
